# baseline (speedup 1.0000x reference)
import functools

import jax
import jax.numpy as jnp
import numpy as np
from jax import lax
from jax.experimental import pallas as pl
from jax.experimental.pallas import tpu as pltpu
from jax.experimental.pallas import tpu_sc as plsc

_NBITS = 2

_ROWS, _COLS = 8192, 2048
_S = 3072
_NC, _NS, _L = 2, 16, 16
_NW = _NC * _NS
_ROWS_W = _S // _NW
_CR = 8
_NCHUNK = _ROWS_W // _CR

_TC_ROWS = _ROWS - _S
_TC_BLOCK = 512


def _enc_matrix():
    bitvecs = np.unpackbits(
        np.arange(2 ** _NBITS, dtype=np.uint8).reshape(-1, 1), axis=1
    )[:, -_NBITS:]
    return jnp.asarray(bitvecs.astype(np.float32) * 2.0 - 1.0)


def _sc_body(p_hbm, x_hbm, o_hbm, p_v, in_v, out_v,
             isem0, isem1, osem0, osem1):
    wid = lax.axis_index("s") * _NC + lax.axis_index("c")
    base = wid * _ROWS_W

    pltpu.sync_copy(p_hbm, p_v)
    lo_mag = p_v[2]
    hi_mag = p_v[3]
    t_hi = p_v[6]
    sign_mask = jnp.full((_L,), jnp.int32(-2147483648))

    isems = (isem0, isem1)
    osems = (osem0, osem1)

    def in_copy(c, slot, sem):
        return pltpu.async_copy(
            x_hbm.at[pl.ds(base + c * _CR, _CR), :], in_v.at[slot], sem)

    def out_copy(c, slot, sem):
        return pltpu.async_copy(
            out_v.at[slot], o_hbm.at[pl.ds(base + c * _CR, _CR), :], sem)

    in_copy(0, 0, isems[0])

    def pair_body(pair, carry):
        for b in range(2):
            c = pair * 2 + b
            pltpu.make_async_copy(
                x_hbm.at[pl.ds(0, _CR), :], in_v.at[b], isems[b]).wait()
            @pl.when(c + 1 < _NCHUNK)
            def _():
                in_copy(c + 1, 1 - b, isems[1 - b])
            @pl.when(c >= 2)
            def _():
                pltpu.make_async_copy(
                    out_v.at[b], o_hbm.at[pl.ds(0, _CR), :], osems[b]).wait()

            for r in range(_CR):
                @plsc.parallel_loop(0, _COLS // _L, unroll=16)
                def _(j):
                    v = in_v[b, r, pl.ds(j * _L, _L)]
                    mag = jnp.where(jnp.abs(v) > t_hi, hi_mag, lo_mag)
                    sbit = lax.bitcast_convert_type(v, jnp.int32) & sign_mask
                    res = lax.bitcast_convert_type(mag, jnp.int32) | sbit
                    out_v[b, r, pl.ds(j * _L, _L)] = (
                        lax.bitcast_convert_type(res, jnp.float32))

            out_copy(c, b, osems[b])
        return carry

    lax.fori_loop(0, _NCHUNK // 2, pair_body, 0)

    for b in range(2):
        pltpu.make_async_copy(
            out_v.at[b], o_hbm.at[pl.ds(0, _CR), :], osems[b]).wait()


def _tc_body(p_ref, x_ref, o_ref):
    v = x_ref[...]
    l0 = p_ref[0]
    l1 = p_ref[1]
    l2 = p_ref[2]
    l3 = p_ref[3]
    t0 = p_ref[4]
    t1 = p_ref[5]
    t2 = p_ref[6]
    lo = jnp.where(v > t0, l1, l0)
    hi = jnp.where(v > t2, l3, l2)
    o_ref[...] = jnp.where(v > t1, hi, lo)


def kernel(x, basis):
    qlevels = jnp.sort(_enc_matrix() @ basis)
    thres = (qlevels[:-1] + qlevels[1:]) * 0.5
    params = jnp.concatenate([qlevels, thres])
    params_sc = jnp.broadcast_to(params[:, None], (7, _L))

    xf = x.reshape(_ROWS, _COLS)
    mesh = plsc.VectorSubcoreMesh(core_axis_name="c", subcore_axis_name="s")

    sc_run = pl.kernel(
        _sc_body,
        mesh=mesh,
        out_type=jax.ShapeDtypeStruct((_S, _COLS), jnp.float32),
        compiler_params=pltpu.CompilerParams(use_tc_tiling_on_sc=True),
        scratch_types=[
            pltpu.VMEM((7, _L), jnp.float32),
            pltpu.VMEM((2, _CR, _COLS), jnp.float32),
            pltpu.VMEM((2, _CR, _COLS), jnp.float32),
            pltpu.SemaphoreType.DMA,
            pltpu.SemaphoreType.DMA,
            pltpu.SemaphoreType.DMA,
            pltpu.SemaphoreType.DMA,
        ],
    )
    sc_out = sc_run(params_sc, xf)

    tc_out = pl.pallas_call(
        _tc_body,
        grid=(_TC_ROWS // _TC_BLOCK,),
        in_specs=[
            pl.BlockSpec(memory_space=pltpu.SMEM),
            pl.BlockSpec((_TC_BLOCK, _COLS),
                         lambda i: (_S // _TC_BLOCK + i, 0)),
        ],
        out_specs=pl.BlockSpec((_TC_BLOCK, _COLS), lambda i: (i, 0)),
        out_shape=jax.ShapeDtypeStruct((_TC_ROWS, _COLS), jnp.float32),
    )(params, xf)

    out = jnp.concatenate([sc_out, tc_out], axis=0)
    return out.reshape(x.shape)


# speedup vs baseline: 1.5189x; 1.5189x over previous
import functools

import jax
import jax.numpy as jnp
import numpy as np
from jax import lax
from jax.experimental import pallas as pl
from jax.experimental.pallas import tpu as pltpu
from jax.experimental.pallas import tpu_sc as plsc

_NBITS = 2

_ROWS, _COLS = 8192, 2048
_NC, _NS, _L = 2, 16, 16
_NW = _NC * _NS
_ROWS_W = _ROWS // _NW
_CR = 4
_NCHUNK = _ROWS_W // _CR
_NBUF = 4


def _enc_matrix():
    bitvecs = np.unpackbits(
        np.arange(2 ** _NBITS, dtype=np.uint8).reshape(-1, 1), axis=1
    )[:, -_NBITS:]
    return jnp.asarray(bitvecs.astype(np.float32) * 2.0 - 1.0)


def _sc_body(p_hbm, x_hbm, o_hbm, p_v, in_v, out_v, *sems):
    isems = sems[:_NBUF]
    osems = sems[_NBUF:]
    wid = lax.axis_index("s") * _NC + lax.axis_index("c")
    base = wid * _ROWS_W

    pltpu.sync_copy(p_hbm, p_v)
    lo_mag = p_v[2]
    hi_mag = p_v[3]
    t_hi = p_v[6]
    sign_mask = jnp.full((_L,), jnp.int32(-2147483648))

    def in_copy(c, slot, sem):
        return pltpu.async_copy(
            x_hbm.at[pl.ds(base + c * _CR, _CR), :], in_v.at[slot], sem)

    def out_copy(c, slot, sem):
        return pltpu.async_copy(
            out_v.at[slot], o_hbm.at[pl.ds(base + c * _CR, _CR), :], sem)

    for k in range(_NBUF - 1):
        in_copy(k, k, isems[k])

    def group_body(g, carry):
        for b in range(_NBUF):
            c = g * _NBUF + b
            pltpu.make_async_copy(
                x_hbm.at[pl.ds(0, _CR), :], in_v.at[b], isems[b]).wait()
            @pl.when(c + _NBUF - 1 < _NCHUNK)
            def _():
                in_copy(c + _NBUF - 1, (b + _NBUF - 1) % _NBUF,
                        isems[(b + _NBUF - 1) % _NBUF])
            @pl.when(c >= _NBUF)
            def _():
                pltpu.make_async_copy(
                    out_v.at[b], o_hbm.at[pl.ds(0, _CR), :], osems[b]).wait()

            for r in range(_CR):
                @plsc.parallel_loop(0, _COLS // _L, unroll=16)
                def _(j):
                    v = in_v[b, r, pl.ds(j * _L, _L)]
                    mag = jnp.where(jnp.abs(v) > t_hi, hi_mag, lo_mag)
                    sbit = lax.bitcast_convert_type(v, jnp.int32) & sign_mask
                    res = lax.bitcast_convert_type(mag, jnp.int32) | sbit
                    out_v[b, r, pl.ds(j * _L, _L)] = (
                        lax.bitcast_convert_type(res, jnp.float32))

            out_copy(c, b, osems[b])
        return carry

    lax.fori_loop(0, _NCHUNK // _NBUF, group_body, 0)

    for b in range(_NBUF):
        pltpu.make_async_copy(
            out_v.at[b], o_hbm.at[pl.ds(0, _CR), :], osems[b]).wait()


def kernel(x, basis):
    qlevels = jnp.sort(_enc_matrix() @ basis)
    thres = (qlevels[:-1] + qlevels[1:]) * 0.5
    params = jnp.broadcast_to(
        jnp.concatenate([qlevels, thres])[:, None], (7, _L))

    xf = x.reshape(_ROWS, _COLS)
    mesh = plsc.VectorSubcoreMesh(core_axis_name="c", subcore_axis_name="s")

    run = pl.kernel(
        _sc_body,
        mesh=mesh,
        out_type=jax.ShapeDtypeStruct((_ROWS, _COLS), jnp.float32),
        compiler_params=pltpu.CompilerParams(use_tc_tiling_on_sc=True),
        scratch_types=[
            pltpu.VMEM((7, _L), jnp.float32),
            pltpu.VMEM((_NBUF, _CR, _COLS), jnp.float32),
            pltpu.VMEM((_NBUF, _CR, _COLS), jnp.float32),
        ] + [pltpu.SemaphoreType.DMA] * (2 * _NBUF),
    )
    out = run(params, xf)
    return out.reshape(x.shape)


# speedup vs baseline: 1.5395x; 1.0136x over previous
import functools

import jax
import jax.numpy as jnp
import numpy as np
from jax import lax
from jax.experimental import pallas as pl
from jax.experimental.pallas import tpu as pltpu
from jax.experimental.pallas import tpu_sc as plsc

_NBITS = 2

_ROWS, _COLS = 8192, 2048
_NC, _NS, _L = 2, 16, 16
_NW = _NC * _NS
_ROWS_W = _ROWS // _NW
_CR = 2
_NCHUNK = _ROWS_W // _CR
_NBUF = 8


def _enc_matrix():
    bitvecs = np.unpackbits(
        np.arange(2 ** _NBITS, dtype=np.uint8).reshape(-1, 1), axis=1
    )[:, -_NBITS:]
    return jnp.asarray(bitvecs.astype(np.float32) * 2.0 - 1.0)


def _sc_body(p_hbm, x_hbm, o_hbm, p_v, in_v, out_v, *sems):
    isems = sems[:_NBUF]
    osems = sems[_NBUF:]
    wid = lax.axis_index("s") * _NC + lax.axis_index("c")
    base = wid * _ROWS_W

    pltpu.sync_copy(p_hbm, p_v)
    lo_mag = p_v[2]
    hi_mag = p_v[3]
    t_hi = p_v[6]
    sign_mask = jnp.full((_L,), jnp.int32(-2147483648))

    def in_copy(c, slot, sem):
        return pltpu.async_copy(
            x_hbm.at[pl.ds(base + c * _CR, _CR), :], in_v.at[slot], sem)

    def out_copy(c, slot, sem):
        return pltpu.async_copy(
            out_v.at[slot], o_hbm.at[pl.ds(base + c * _CR, _CR), :], sem)

    for k in range(_NBUF - 1):
        in_copy(k, k, isems[k])

    def group_body(g, carry):
        for b in range(_NBUF):
            c = g * _NBUF + b
            pltpu.make_async_copy(
                x_hbm.at[pl.ds(0, _CR), :], in_v.at[b], isems[b]).wait()
            @pl.when(c + _NBUF - 1 < _NCHUNK)
            def _():
                in_copy(c + _NBUF - 1, (b + _NBUF - 1) % _NBUF,
                        isems[(b + _NBUF - 1) % _NBUF])
            @pl.when(c >= _NBUF)
            def _():
                pltpu.make_async_copy(
                    out_v.at[b], o_hbm.at[pl.ds(0, _CR), :], osems[b]).wait()

            for r in range(_CR):
                @plsc.parallel_loop(0, _COLS // _L, unroll=16)
                def _(j):
                    v = in_v[b, r, pl.ds(j * _L, _L)]
                    mag = jnp.where(jnp.abs(v) > t_hi, hi_mag, lo_mag)
                    sbit = lax.bitcast_convert_type(v, jnp.int32) & sign_mask
                    res = lax.bitcast_convert_type(mag, jnp.int32) | sbit
                    out_v[b, r, pl.ds(j * _L, _L)] = (
                        lax.bitcast_convert_type(res, jnp.float32))

            out_copy(c, b, osems[b])
        return carry

    lax.fori_loop(0, _NCHUNK // _NBUF, group_body, 0)

    for b in range(_NBUF):
        pltpu.make_async_copy(
            out_v.at[b], o_hbm.at[pl.ds(0, _CR), :], osems[b]).wait()


def kernel(x, basis):
    qlevels = jnp.sort(_enc_matrix() @ basis)
    thres = (qlevels[:-1] + qlevels[1:]) * 0.5
    params = jnp.broadcast_to(
        jnp.concatenate([qlevels, thres])[:, None], (7, _L))

    xf = x.reshape(_ROWS, _COLS)
    mesh = plsc.VectorSubcoreMesh(core_axis_name="c", subcore_axis_name="s")

    run = pl.kernel(
        _sc_body,
        mesh=mesh,
        out_type=jax.ShapeDtypeStruct((_ROWS, _COLS), jnp.float32),
        compiler_params=pltpu.CompilerParams(use_tc_tiling_on_sc=True),
        scratch_types=[
            pltpu.VMEM((7, _L), jnp.float32),
            pltpu.VMEM((_NBUF, _CR, _COLS), jnp.float32),
            pltpu.VMEM((_NBUF, _CR, _COLS), jnp.float32),
        ] + [pltpu.SemaphoreType.DMA] * (2 * _NBUF),
    )
    out = run(params, xf)
    return out.reshape(x.shape)


# speedup vs baseline: 1.5428x; 1.0021x over previous
import functools

import jax
import jax.numpy as jnp
import numpy as np
from jax import lax
from jax.experimental import pallas as pl
from jax.experimental.pallas import tpu as pltpu
from jax.experimental.pallas import tpu_sc as plsc

_NBITS = 2

_ROWS, _COLS = 8192, 2048
_NC, _NS, _L = 2, 16, 16
_NW = _NC * _NS
_ROWS_W = _ROWS // _NW
_CR = 2
_NCHUNK = _ROWS_W // _CR
_NBUF = 8


def _enc_matrix():
    bitvecs = np.unpackbits(
        np.arange(2 ** _NBITS, dtype=np.uint8).reshape(-1, 1), axis=1
    )[:, -_NBITS:]
    return jnp.asarray(bitvecs.astype(np.float32) * 2.0 - 1.0)


def _sc_body(p_hbm, x_hbm, o_hbm, p_v, in_v, out_v, *sems):
    isems = sems[:_NBUF]
    osems = sems[_NBUF:]
    wid = lax.axis_index("s") * _NC + lax.axis_index("c")
    base = wid * _ROWS_W

    pltpu.sync_copy(p_hbm, p_v)
    av = jnp.abs(p_v[0])
    hi_mag = plsc.cummax(lax.rev(plsc.cumsum(av), (0,)))
    t_hi = plsc.cummax(lax.rev(plsc.cummax(av), (0,)))
    lo_mag = 2.0 * t_hi - hi_mag
    sign_mask = jnp.full((_L,), jnp.int32(-2147483648))

    def in_copy(c, slot, sem):
        return pltpu.async_copy(
            x_hbm.at[pl.ds(base + c * _CR, _CR), :], in_v.at[slot], sem)

    def out_copy(c, slot, sem):
        return pltpu.async_copy(
            out_v.at[slot], o_hbm.at[pl.ds(base + c * _CR, _CR), :], sem)

    for k in range(_NBUF - 1):
        in_copy(k, k, isems[k])

    def group_body(g, carry):
        for b in range(_NBUF):
            c = g * _NBUF + b
            pltpu.make_async_copy(
                x_hbm.at[pl.ds(0, _CR), :], in_v.at[b], isems[b]).wait()
            @pl.when(c + _NBUF - 1 < _NCHUNK)
            def _():
                in_copy(c + _NBUF - 1, (b + _NBUF - 1) % _NBUF,
                        isems[(b + _NBUF - 1) % _NBUF])
            @pl.when(c >= _NBUF)
            def _():
                pltpu.make_async_copy(
                    out_v.at[b], o_hbm.at[pl.ds(0, _CR), :], osems[b]).wait()

            for r in range(_CR):
                @plsc.parallel_loop(0, _COLS // _L, unroll=16)
                def _(j):
                    v = in_v[b, r, pl.ds(j * _L, _L)]
                    mag = jnp.where(jnp.abs(v) > t_hi, hi_mag, lo_mag)
                    sbit = lax.bitcast_convert_type(v, jnp.int32) & sign_mask
                    res = lax.bitcast_convert_type(mag, jnp.int32) | sbit
                    out_v[b, r, pl.ds(j * _L, _L)] = (
                        lax.bitcast_convert_type(res, jnp.float32))

            out_copy(c, b, osems[b])
        return carry

    lax.fori_loop(0, _NCHUNK // _NBUF, group_body, 0)

    for b in range(_NBUF):
        pltpu.make_async_copy(
            out_v.at[b], o_hbm.at[pl.ds(0, _CR), :], osems[b]).wait()


def kernel(x, basis):
    params = jnp.pad(basis.astype(jnp.float32), (0, _L - 2)).reshape(1, _L)
    xf = x.reshape(_ROWS, _COLS)
    mesh = plsc.VectorSubcoreMesh(core_axis_name="c", subcore_axis_name="s")

    run = pl.kernel(
        _sc_body,
        mesh=mesh,
        out_type=jax.ShapeDtypeStruct((_ROWS, _COLS), jnp.float32),
        compiler_params=pltpu.CompilerParams(
            use_tc_tiling_on_sc=True, needs_layout_passes=False),
        scratch_types=[
            pltpu.VMEM((1, _L), jnp.float32),
            pltpu.VMEM((_NBUF, _CR, _COLS), jnp.float32),
            pltpu.VMEM((_NBUF, _CR, _COLS), jnp.float32),
        ] + [pltpu.SemaphoreType.DMA] * (2 * _NBUF),
    )
    out = run(params, xf)
    return out.reshape(x.shape)


# speedup vs baseline: 1.5456x; 1.0018x over previous
import functools

import jax
import jax.numpy as jnp
import numpy as np
from jax import lax
from jax.experimental import pallas as pl
from jax.experimental.pallas import tpu as pltpu
from jax.experimental.pallas import tpu_sc as plsc

_NBITS = 2

_ROWS, _COLS = 8192, 2048
_NC, _NS, _L = 2, 16, 16
_NW = _NC * _NS
_ROWS_W = _ROWS // _NW
_CR = 2
_NCHUNK = _ROWS_W // _CR
_NBUF = 8


def _enc_matrix():
    bitvecs = np.unpackbits(
        np.arange(2 ** _NBITS, dtype=np.uint8).reshape(-1, 1), axis=1
    )[:, -_NBITS:]
    return jnp.asarray(bitvecs.astype(np.float32) * 2.0 - 1.0)


def _sc_body(p_hbm, x_hbm, o_hbm, p_v, in_v, out_v, *sems):
    isems = sems[:_NBUF]
    osems = sems[_NBUF:]
    wid = lax.axis_index("s") * _NC + lax.axis_index("c")
    base = wid * _ROWS_W

    pltpu.sync_copy(p_hbm, p_v)
    lo_mag = p_v[0]
    hi_mag = p_v[1]
    t_hi = p_v[2]
    sign_mask = jnp.full((_L,), jnp.int32(-2147483648))

    def in_copy(c, slot, sem):
        return pltpu.async_copy(
            x_hbm.at[pl.ds(base + c * _CR, _CR), :], in_v.at[slot], sem)

    def out_copy(c, slot, sem):
        return pltpu.async_copy(
            out_v.at[slot], o_hbm.at[pl.ds(base + c * _CR, _CR), :], sem)

    for k in range(_NBUF - 1):
        in_copy(k, k, isems[k])

    def group_body(g, carry):
        for b in range(_NBUF):
            c = g * _NBUF + b
            pltpu.make_async_copy(
                x_hbm.at[pl.ds(0, _CR), :], in_v.at[b], isems[b]).wait()
            @pl.when(c + _NBUF - 1 < _NCHUNK)
            def _():
                in_copy(c + _NBUF - 1, (b + _NBUF - 1) % _NBUF,
                        isems[(b + _NBUF - 1) % _NBUF])
            @pl.when(c >= _NBUF)
            def _():
                pltpu.make_async_copy(
                    out_v.at[b], o_hbm.at[pl.ds(0, _CR), :], osems[b]).wait()

            for r in range(_CR):
                @plsc.parallel_loop(0, _COLS // _L, unroll=16)
                def _(j):
                    v = in_v[b, r, pl.ds(j * _L, _L)]
                    mag = jnp.where(jnp.abs(v) > t_hi, hi_mag, lo_mag)
                    sbit = lax.bitcast_convert_type(v, jnp.int32) & sign_mask
                    res = lax.bitcast_convert_type(mag, jnp.int32) | sbit
                    out_v[b, r, pl.ds(j * _L, _L)] = (
                        lax.bitcast_convert_type(res, jnp.float32))

            out_copy(c, b, osems[b])
        return carry

    lax.fori_loop(0, _NCHUNK // _NBUF, group_body, 0)

    for b in range(_NBUF):
        pltpu.make_async_copy(
            out_v.at[b], o_hbm.at[pl.ds(0, _CR), :], osems[b]).wait()


def kernel(x, basis):
    a0 = jnp.abs(basis[0])
    a1 = jnp.abs(basis[1])
    params = jnp.broadcast_to(
        jnp.stack([jnp.abs(a0 - a1), a0 + a1, jnp.maximum(a0, a1)])[:, None],
        (3, _L))
    xf = x.reshape(_ROWS, _COLS)
    mesh = plsc.VectorSubcoreMesh(core_axis_name="c", subcore_axis_name="s")

    run = pl.kernel(
        _sc_body,
        mesh=mesh,
        out_type=jax.ShapeDtypeStruct((_ROWS, _COLS), jnp.float32),
        compiler_params=pltpu.CompilerParams(use_tc_tiling_on_sc=True),
        scratch_types=[
            pltpu.VMEM((3, _L), jnp.float32),
            pltpu.VMEM((_NBUF, _CR, _COLS), jnp.float32),
            pltpu.VMEM((_NBUF, _CR, _COLS), jnp.float32),
        ] + [pltpu.SemaphoreType.DMA] * (2 * _NBUF),
    )
    out = run(params, xf)
    return out.reshape(x.shape)
